# baseline (speedup 1.0000x reference)
import functools

import jax
import jax.numpy as jnp
from jax import lax
from jax.experimental import pallas as pl
from jax.experimental.pallas import tpu as pltpu
from jax.experimental.pallas import tpu_sc as plsc

DIM_ = 64
NC_ = 2
NS_ = 16
NW_ = NC_ * NS_
V_ = 1000000

TC_ = 64
GRP_ = 20
NBUF_ = 10
DEPTH_ = 5


@functools.partial(jax.jit, static_argnames=("n_groups",))
def _gather_call(idx4, table, *, n_groups):
    n_chunks = n_groups * GRP_
    B = NW_ * n_chunks * TC_
    mesh = plsc.VectorSubcoreMesh(core_axis_name="c", subcore_axis_name="s")

    sem_types = [pltpu.SemaphoreType.DMA] * (2 * NBUF_ + 1)

    @functools.partial(
        pl.kernel,
        mesh=mesh,
        out_type=jax.ShapeDtypeStruct((2 * B, 32), jnp.float32),
        scratch_types=[
            pltpu.VMEM((2, GRP_, 2 * TC_), jnp.int32),
            pltpu.VMEM((NBUF_, 2 * TC_, 32), jnp.float32),
        ] + sem_types,
        compiler_params=pltpu.CompilerParams(use_tc_tiling_on_sc=False),
    )
    def k(idx_hbm, table_hbm, out_hbm, idx_v, rows_v, *sems):
        gsem = sems[:NBUF_]
        ssem = sems[NBUF_:2 * NBUF_]
        isem = sems[2 * NBUF_:]
        wid = lax.axis_index("s") * NC_ + lax.axis_index("c")
        base = wid * n_chunks * TC_

        def idx_copy(g, gb):
            return pltpu.make_async_copy(idx_hbm.at[wid, g], idx_v.at[gb],
                                         isem[0])

        def start_gather(gb, j, b):
            pltpu.async_copy(table_hbm.at[idx_v.at[gb, j]], rows_v.at[b],
                             gsem[b])

        def wait_gather(b):
            pltpu.make_async_copy(table_hbm.at[idx_v.at[0, 0]], rows_v.at[b],
                                  gsem[b]).wait()

        def start_store(s, b):
            tok0 = base + s * TC_
            pltpu.async_copy(
                rows_v.at[b],
                out_hbm.at[pl.ds(2 * tok0, 2 * TC_)],
                ssem[b])

        def wait_store(b):
            pltpu.make_async_copy(
                rows_v.at[b],
                out_hbm.at[pl.ds(0, 2 * TC_)],
                ssem[b]).wait()

        pltpu.sync_copy(idx_hbm.at[wid, 0], idx_v.at[0])
        for j in range(DEPTH_):
            start_gather(0, j, j % NBUF_)

        def group_body(g, carry):
            gb_cur = g % 2
            gb_nxt = (g + 1) % 2
            for j in range(GRP_):
                s = g * GRP_ + j
                b = j % NBUF_

                if j == 0:
                    @pl.when(g + 1 < n_groups)
                    def _():
                        idx_copy(g + 1, gb_nxt).start()

                wait_gather(b)
                start_store(s, b)

                nxt_j = j + DEPTH_
                b2 = nxt_j % NBUF_

                @pl.when(s + DEPTH_ >= NBUF_)
                def _():
                    wait_store(b2)

                if j == GRP_ - DEPTH_:
                    @pl.when(g + 1 < n_groups)
                    def _():
                        idx_copy(g + 1, gb_nxt).wait()

                if nxt_j < GRP_:
                    @pl.when(s + DEPTH_ < n_chunks)
                    def _():
                        start_gather(gb_cur, nxt_j, b2)
                else:
                    @pl.when(s + DEPTH_ < n_chunks)
                    def _():
                        start_gather(gb_nxt, nxt_j - GRP_, b2)
            return carry

        lax.fori_loop(0, n_groups, group_body, 0)

        for i in range(DEPTH_):
            wait_store((n_chunks - DEPTH_ + i) % NBUF_)

    return k(idx4, table)


def kernel(x, emb_weight):
    B = x.shape[0] * x.shape[1]
    n_groups = B // (NW_ * GRP_ * TC_)
    c2 = x.reshape(-1, TC_).astype(jnp.int32)
    idx2 = jnp.stack([2 * c2, 2 * c2 + 1], axis=2).reshape(-1, 2 * TC_)
    idx4 = idx2.reshape(NW_, n_groups, GRP_, 2 * TC_)
    table500 = lax.optimization_barrier(emb_weight.reshape(V_ // 2, 128))
    table2m = table500.reshape(2 * V_, 32)
    out = _gather_call(idx4, table2m, n_groups=n_groups)
    return out.reshape(x.shape[0], x.shape[1], DIM_)


# speedup vs baseline: 1.4609x; 1.4609x over previous
import functools

import jax
import jax.numpy as jnp
from jax import lax
from jax.experimental import pallas as pl
from jax.experimental.pallas import tpu as pltpu
from jax.experimental.pallas import tpu_sc as plsc

DIM_ = 64
NC_ = 2
NS_ = 16
NW_ = NC_ * NS_
V_ = 1000000
PADD_ = 128

CHUNK_ = 128
GRP_ = 20
NBUF_ = 10
DEPTH_ = 5


@functools.partial(jax.jit, static_argnames=("n_groups",))
def _gather_call(idx4, table, *, n_groups):
    n_chunks = n_groups * GRP_
    B = NW_ * n_chunks * CHUNK_
    mesh = plsc.VectorSubcoreMesh(core_axis_name="c", subcore_axis_name="s")

    sem_types = [pltpu.SemaphoreType.DMA] * (2 * NBUF_ + 1)

    @functools.partial(
        pl.kernel,
        mesh=mesh,
        out_type=jax.ShapeDtypeStruct((B, PADD_), jnp.float32),
        scratch_types=[
            pltpu.VMEM((2, GRP_, CHUNK_), jnp.int32),
            pltpu.VMEM((NBUF_, CHUNK_, DIM_), jnp.float32),
        ] + sem_types,
        compiler_params=pltpu.CompilerParams(use_tc_tiling_on_sc=False),
    )
    def k(idx_hbm, table_hbm, out_hbm, idx_v, rows_v, *sems):
        gsem = sems[:NBUF_]
        ssem = sems[NBUF_:2 * NBUF_]
        isem = sems[2 * NBUF_:]
        wid = lax.axis_index("s") * NC_ + lax.axis_index("c")
        base = wid * n_chunks * CHUNK_

        def idx_copy(g, gb):
            return pltpu.make_async_copy(idx_hbm.at[wid, g], idx_v.at[gb],
                                         isem[0])

        def start_gather(gb, j, b):
            pltpu.async_copy(table_hbm.at[idx_v.at[gb, j]], rows_v.at[b],
                             gsem[b])

        def wait_gather(b):
            pltpu.make_async_copy(table_hbm.at[idx_v.at[0, 0]], rows_v.at[b],
                                  gsem[b]).wait()

        def start_store(s, b):
            pltpu.async_copy(
                rows_v.at[b],
                out_hbm.at[pl.ds(base + s * CHUNK_, CHUNK_), pl.ds(0, DIM_)],
                ssem[b])

        def wait_store(b):
            pltpu.make_async_copy(
                rows_v.at[b],
                out_hbm.at[pl.ds(0, CHUNK_), pl.ds(0, DIM_)],
                ssem[b]).wait()

        pltpu.sync_copy(idx_hbm.at[wid, 0], idx_v.at[0])
        for j in range(DEPTH_):
            start_gather(0, j, j % NBUF_)

        def group_body(g, carry):
            gb_cur = g % 2
            gb_nxt = (g + 1) % 2
            for j in range(GRP_):
                s = g * GRP_ + j
                b = j % NBUF_

                if j == 0:
                    @pl.when(g + 1 < n_groups)
                    def _():
                        idx_copy(g + 1, gb_nxt).start()

                wait_gather(b)
                start_store(s, b)

                nxt_j = j + DEPTH_
                b2 = nxt_j % NBUF_

                @pl.when(s + DEPTH_ >= NBUF_)
                def _():
                    wait_store(b2)

                if j == GRP_ - DEPTH_:
                    @pl.when(g + 1 < n_groups)
                    def _():
                        idx_copy(g + 1, gb_nxt).wait()

                if nxt_j < GRP_:
                    @pl.when(s + DEPTH_ < n_chunks)
                    def _():
                        start_gather(gb_cur, nxt_j, b2)
                else:
                    @pl.when(s + DEPTH_ < n_chunks)
                    def _():
                        start_gather(gb_nxt, nxt_j - GRP_, b2)
            return carry

        lax.fori_loop(0, n_groups, group_body, 0)

        for i in range(DEPTH_):
            wait_store((n_chunks - DEPTH_ + i) % NBUF_)

    return k(idx4, table)


def kernel(x, emb_weight):
    B = x.shape[0] * x.shape[1]
    n_groups = B // (NW_ * GRP_ * CHUNK_)
    idx4 = x.reshape(NW_, n_groups, GRP_, CHUNK_).astype(jnp.int32)
    out = _gather_call(idx4, emb_weight, n_groups=n_groups)
    return out[:, :DIM_].reshape(x.shape[0], x.shape[1], DIM_)


# speedup vs baseline: 1.4623x; 1.0009x over previous
import functools

import jax
import jax.numpy as jnp
from jax import lax
from jax.experimental import pallas as pl
from jax.experimental.pallas import tpu as pltpu
from jax.experimental.pallas import tpu_sc as plsc

DIM_ = 64
NC_ = 2
NS_ = 16
NW_ = NC_ * NS_
V_ = 1000000
PADD_ = 128

CHUNK_ = 128
GRP_ = 20
NBUF_ = 10
DEPTH_ = 7


@functools.partial(jax.jit, static_argnames=("n_groups",))
def _gather_call(idx4, table, *, n_groups):
    n_chunks = n_groups * GRP_
    B = NW_ * n_chunks * CHUNK_
    mesh = plsc.VectorSubcoreMesh(core_axis_name="c", subcore_axis_name="s")

    sem_types = [pltpu.SemaphoreType.DMA] * (2 * NBUF_ + 1)

    @functools.partial(
        pl.kernel,
        mesh=mesh,
        out_type=jax.ShapeDtypeStruct((B, PADD_), jnp.float32),
        scratch_types=[
            pltpu.VMEM((2, GRP_, CHUNK_), jnp.int32),
            pltpu.VMEM((NBUF_, CHUNK_, DIM_), jnp.float32),
        ] + sem_types,
        compiler_params=pltpu.CompilerParams(use_tc_tiling_on_sc=False),
    )
    def k(idx_hbm, table_hbm, out_hbm, idx_v, rows_v, *sems):
        gsem = sems[:NBUF_]
        ssem = sems[NBUF_:2 * NBUF_]
        isem = sems[2 * NBUF_:]
        wid = lax.axis_index("s") * NC_ + lax.axis_index("c")
        base = wid * n_chunks * CHUNK_

        def idx_copy(g, gb):
            return pltpu.make_async_copy(idx_hbm.at[wid, g], idx_v.at[gb],
                                         isem[0])

        def start_gather(gb, j, b):
            pltpu.async_copy(table_hbm.at[idx_v.at[gb, j]], rows_v.at[b],
                             gsem[b])

        def wait_gather(b):
            pltpu.make_async_copy(table_hbm.at[idx_v.at[0, 0]], rows_v.at[b],
                                  gsem[b]).wait()

        def start_store(s, b):
            pltpu.async_copy(
                rows_v.at[b],
                out_hbm.at[pl.ds(base + s * CHUNK_, CHUNK_), pl.ds(0, DIM_)],
                ssem[b])

        def wait_store(b):
            pltpu.make_async_copy(
                rows_v.at[b],
                out_hbm.at[pl.ds(0, CHUNK_), pl.ds(0, DIM_)],
                ssem[b]).wait()

        pltpu.sync_copy(idx_hbm.at[wid, 0], idx_v.at[0])
        for j in range(DEPTH_):
            start_gather(0, j, j % NBUF_)

        def group_body(g, carry):
            gb_cur = g % 2
            gb_nxt = (g + 1) % 2
            for j in range(GRP_):
                s = g * GRP_ + j
                b = j % NBUF_

                if j == 0:
                    @pl.when(g + 1 < n_groups)
                    def _():
                        idx_copy(g + 1, gb_nxt).start()

                wait_gather(b)
                start_store(s, b)

                nxt_j = j + DEPTH_
                b2 = nxt_j % NBUF_

                @pl.when(s + DEPTH_ >= NBUF_)
                def _():
                    wait_store(b2)

                if j == GRP_ - DEPTH_:
                    @pl.when(g + 1 < n_groups)
                    def _():
                        idx_copy(g + 1, gb_nxt).wait()

                if nxt_j < GRP_:
                    @pl.when(s + DEPTH_ < n_chunks)
                    def _():
                        start_gather(gb_cur, nxt_j, b2)
                else:
                    @pl.when(s + DEPTH_ < n_chunks)
                    def _():
                        start_gather(gb_nxt, nxt_j - GRP_, b2)
            return carry

        lax.fori_loop(0, n_groups, group_body, 0)

        und = NBUF_ - DEPTH_
        for i in range(und):
            wait_store((n_chunks - und + i) % NBUF_)

    return k(idx4, table)


def kernel(x, emb_weight):
    B = x.shape[0] * x.shape[1]
    n_groups = B // (NW_ * GRP_ * CHUNK_)
    idx4 = x.reshape(NW_, n_groups, GRP_, CHUNK_).astype(jnp.int32)
    out = _gather_call(idx4, emb_weight, n_groups=n_groups)
    return out[:, :DIM_].reshape(x.shape[0], x.shape[1], DIM_)
